# Initial kernel scaffold; baseline (speedup 1.0000x reference)
import jax
import jax.numpy as jnp
from jax.experimental import pallas as pl


def kernel(x, table):
    raise NotImplementedError("write your pallas kernel here")


# speedup vs baseline: 4.8091x; 4.8091x over previous
import functools

import jax
import jax.numpy as jnp
from jax import lax
from jax.experimental import pallas as pl
from jax.experimental.pallas import tpu as pltpu
from jax.experimental.pallas import tpu_sc as plsc


@functools.cache
def _make(B, V, D):
    info = plsc.get_sparse_core_info()
    NW = info.num_cores * info.num_subcores
    b_per_w = B // NW
    C = 1024
    n_chunks = b_per_w // C
    mesh = plsc.VectorSubcoreMesh(core_axis_name="c", subcore_axis_name="s")

    @functools.partial(
        pl.kernel,
        mesh=mesh,
        compiler_params=pltpu.CompilerParams(use_tc_tiling_on_sc=False),
        out_type=jax.ShapeDtypeStruct((B, D), jnp.float32),
        scratch_types=[
            pltpu.VMEM((C,), jnp.int32),
            pltpu.VMEM((C, D), jnp.float32),
            pltpu.SemaphoreType.DMA,
        ],
    )
    def k(x_hbm, table_hbm, out_hbm, idx_v, rows_v, sem):
        wid = lax.axis_index("s") * info.num_cores + lax.axis_index("c")
        base = wid * b_per_w

        def body(i, carry):
            cb = base + i * C
            pltpu.sync_copy(x_hbm.at[pl.ds(cb, C)], idx_v)
            pltpu.async_copy(table_hbm.at[idx_v], rows_v, sem).wait()
            pltpu.sync_copy(rows_v, out_hbm.at[pl.ds(cb, C)])
            return carry

        lax.fori_loop(0, n_chunks, body, 0)

    return k


def kernel(x, table):
    R, S = x.shape
    V, D = table.shape
    B = R * S
    xf = x.reshape(B).astype(jnp.int32)
    out = _make(B, V, D)(xf, table)
    return out.reshape(R, S, D)


# speedup vs baseline: 4.9430x; 1.0278x over previous
import functools

import jax
import jax.numpy as jnp
from jax import lax
from jax.experimental import pallas as pl
from jax.experimental.pallas import tpu as pltpu
from jax.experimental.pallas import tpu_sc as plsc


@functools.cache
def _make(B, V, D):
    info = plsc.get_sparse_core_info()
    NW = info.num_cores * info.num_subcores
    b_per_w = B // NW
    C = 1024
    n_chunks = b_per_w // C
    SUP = 20
    SUPI = SUP * C
    n_super = n_chunks // SUP
    mesh = plsc.VectorSubcoreMesh(core_axis_name="c", subcore_axis_name="s")

    @functools.partial(
        pl.kernel,
        mesh=mesh,
        compiler_params=pltpu.CompilerParams(use_tc_tiling_on_sc=False),
        out_type=jax.ShapeDtypeStruct((B, D), jnp.float32),
        scratch_types=[
            pltpu.VMEM((2 * SUPI,), jnp.int32),
            pltpu.VMEM((C, D), jnp.float32),
            pltpu.VMEM((C, D), jnp.float32),
            pltpu.SemaphoreType.DMA,
            pltpu.SemaphoreType.DMA,
            pltpu.SemaphoreType.DMA,
            pltpu.SemaphoreType.DMA,
            pltpu.SemaphoreType.DMA,
        ],
    )
    def k(x_hbm, table_hbm, out_hbm, idx_v, rows0, rows1, g0, g1, s0, s1, isem):
        wid = lax.axis_index("s") * info.num_cores + lax.axis_index("c")
        base = wid * b_per_w
        rows = (rows0, rows1)
        gsem = (g0, g1)
        ssem = (s0, s1)

        def gwait(p):
            pltpu.make_async_copy(table_hbm.at[pl.ds(0, C)], rows[p], gsem[p]).wait()

        def swait(p):
            pltpu.make_async_copy(rows[p], out_hbm.at[pl.ds(0, C)], ssem[p]).wait()

        def iwait():
            pltpu.make_async_copy(
                x_hbm.at[pl.ds(0, SUPI)], idx_v.at[pl.ds(0, SUPI)], isem).wait()

        pltpu.sync_copy(x_hbm.at[pl.ds(base, SUPI)], idx_v.at[pl.ds(0, SUPI)])

        def body(io, carry):
            for b in range(2):
                i = io * 2 + b
                s = i // SUP
                o = i - s * SUP
                sp = lax.rem(s, 2)
                at_sup = o == 0

                @pl.when(jnp.logical_and(at_sup, s >= 1))
                def _():
                    iwait()

                @pl.when(i >= 2)
                def _():
                    swait(b)

                ids = idx_v.at[pl.ds(sp * SUPI + o * C, C)]
                pltpu.async_copy(table_hbm.at[ids], rows[b], gsem[b])

                @pl.when(i >= 1)
                def _():
                    q = 1 - b
                    gwait(q)
                    pltpu.async_copy(
                        rows[q], out_hbm.at[pl.ds(base + (i - 1) * C, C)], ssem[q])

                @pl.when(jnp.logical_and(at_sup, s + 1 < n_super))
                def _():
                    nsp = 1 - sp
                    pltpu.async_copy(
                        x_hbm.at[pl.ds(base + (s + 1) * SUPI, SUPI)],
                        idx_v.at[pl.ds(nsp * SUPI, SUPI)], isem)
            return carry

        lax.fori_loop(0, n_chunks // 2, body, 0)
        gwait(1)
        pltpu.async_copy(
            rows[1], out_hbm.at[pl.ds(base + (n_chunks - 1) * C, C)], ssem[1])
        swait(0)
        swait(1)

    return k


def kernel(x, table):
    R, S = x.shape
    V, D = table.shape
    B = R * S
    xf = x.reshape(B).astype(jnp.int32)
    out = _make(B, V, D)(xf, table)
    return out.reshape(R, S, D)


# speedup vs baseline: 5.0515x; 1.0219x over previous
import functools

import jax
import jax.numpy as jnp
from jax import lax
from jax.experimental import pallas as pl
from jax.experimental.pallas import tpu as pltpu
from jax.experimental.pallas import tpu_sc as plsc


@functools.cache
def _make(R, S, V, D):
    info = plsc.get_sparse_core_info()
    NW = info.num_cores * info.num_subcores
    r_per_w = R // NW
    SC = 4
    C = SC * S
    n_chunks = r_per_w // SC
    SUP = 8
    SUPI = SUP * C
    n_super = n_chunks // SUP
    NB = 4
    L = NB - 1
    mesh = plsc.VectorSubcoreMesh(core_axis_name="c", subcore_axis_name="s")

    @functools.partial(
        pl.kernel,
        mesh=mesh,
        compiler_params=pltpu.CompilerParams(use_tc_tiling_on_sc=False),
        out_type=jax.ShapeDtypeStruct((R, S, D), jnp.float32),
        scratch_types=[
            pltpu.VMEM((2 * SUPI,), jnp.int32),
            *[pltpu.VMEM((C, D), jnp.float32) for _ in range(NB)],
            *[pltpu.SemaphoreType.DMA for _ in range(NB)],
            *[pltpu.SemaphoreType.DMA for _ in range(NB)],
            pltpu.SemaphoreType.DMA,
        ],
    )
    def k(x_hbm, table_hbm, out_hbm, idx_v, *refs):
        rows = refs[:NB]
        gsem = refs[NB:2 * NB]
        ssem = refs[2 * NB:3 * NB]
        isem = refs[3 * NB]
        wid = lax.axis_index("s") * info.num_cores + lax.axis_index("c")
        base = wid * r_per_w

        def store(i, p):
            for t in range(SC):
                pltpu.async_copy(
                    rows[p].at[pl.ds(t * S, S)],
                    out_hbm.at[base + i * SC + t], ssem[p])

        def gwait(p):
            pltpu.make_async_copy(table_hbm.at[pl.ds(0, C)], rows[p], gsem[p]).wait()

        def swait(p):
            for _ in range(SC):
                pltpu.make_async_copy(
                    rows[p].at[pl.ds(0, S)], out_hbm.at[0], ssem[p]).wait()

        def iwait():
            pltpu.make_async_copy(
                x_hbm.at[pl.ds(0, SUPI)], idx_v.at[pl.ds(0, SUPI)], isem).wait()

        pltpu.sync_copy(
            x_hbm.at[pl.ds(base * S, SUPI)], idx_v.at[pl.ds(0, SUPI)])

        def body(io, carry):
            for b in range(NB):
                i = io * NB + b
                s = i // SUP
                o = i - s * SUP
                sp = lax.rem(s, 2)
                at_sup = o == 0

                @pl.when(jnp.logical_and(at_sup, s >= 1))
                def _():
                    iwait()

                @pl.when(i >= NB)
                def _():
                    swait(b)

                ids = idx_v.at[pl.ds(sp * SUPI + o * C, C)]
                pltpu.async_copy(table_hbm.at[ids], rows[b], gsem[b])

                @pl.when(i >= L)
                def _():
                    q = (b - L) % NB
                    gwait(q)
                    store(i - L, q)

                @pl.when(jnp.logical_and(at_sup, s + 1 < n_super))
                def _():
                    nsp = 1 - sp
                    pltpu.async_copy(
                        x_hbm.at[pl.ds((base + (s + 1) * SUP * SC) * S, SUPI)],
                        idx_v.at[pl.ds(nsp * SUPI, SUPI)], isem)
            return carry

        lax.fori_loop(0, n_chunks // NB, body, 0)
        for t in range(L):
            j = n_chunks - L + t
            q = j % NB
            gwait(q)
            store(j, q)
        for b in range(NB):
            swait(b)

    return k


def kernel(x, table):
    R, S = x.shape
    V, D = table.shape
    xf = x.reshape(R * S).astype(jnp.int32)
    return _make(R, S, V, D)(xf, table)


# speedup vs baseline: 5.0654x; 1.0027x over previous
import functools

import jax
import jax.numpy as jnp
from jax import lax
from jax.experimental import pallas as pl
from jax.experimental.pallas import tpu as pltpu
from jax.experimental.pallas import tpu_sc as plsc


@functools.cache
def _make(R, S, V, D):
    info = plsc.get_sparse_core_info()
    NW = info.num_cores * info.num_subcores
    r_per_w = R // NW
    SC = 4
    C = SC * S
    n_chunks = r_per_w // SC
    SUP = 8
    SUPI = SUP * C
    n_super = n_chunks // SUP
    NB = 4
    L = NB - 1
    mesh = plsc.VectorSubcoreMesh(core_axis_name="c", subcore_axis_name="s")

    @functools.partial(
        pl.kernel,
        mesh=mesh,
        compiler_params=pltpu.CompilerParams(use_tc_tiling_on_sc=False),
        out_type=jax.ShapeDtypeStruct((R, S, D), jnp.float32),
        scratch_types=[
            pltpu.VMEM((2 * SUPI,), jnp.int32),
            *[pltpu.VMEM((C, D), jnp.float32) for _ in range(NB)],
            *[pltpu.SemaphoreType.DMA for _ in range(NB)],
            *[pltpu.SemaphoreType.DMA for _ in range(NB)],
            pltpu.SemaphoreType.DMA,
        ],
    )
    def k(x_hbm, table_hbm, out_hbm, idx_v, *refs):
        rows = refs[:NB]
        gsem = refs[NB:2 * NB]
        ssem = refs[2 * NB:3 * NB]
        isem = refs[3 * NB]
        wid = lax.axis_index("s") * info.num_cores + lax.axis_index("c")
        base = wid * r_per_w

        def store(i, p):
            for t in range(SC):
                pltpu.async_copy(
                    rows[p].at[pl.ds(t * S, S)],
                    out_hbm.at[base + i * SC + t], ssem[p])

        def gwait(p):
            pltpu.make_async_copy(table_hbm.at[pl.ds(0, C)], rows[p], gsem[p]).wait()

        def swait(p):
            for _ in range(SC):
                pltpu.make_async_copy(
                    rows[p].at[pl.ds(0, S)], out_hbm.at[0], ssem[p]).wait()

        def iwait():
            pltpu.make_async_copy(
                x_hbm.at[pl.ds(0, SUPI)], idx_v.at[pl.ds(0, SUPI)], isem).wait()

        pltpu.sync_copy(
            x_hbm.at[pl.ds(base * S, SUPI)], idx_v.at[pl.ds(0, SUPI)])

        def body(io, carry):
            for b in range(NB):
                i = io * NB + b
                s = i // SUP
                o = i - s * SUP
                sp = lax.rem(s, 2)
                at_sup = o == 0

                @pl.when(jnp.logical_and(at_sup, s >= 1))
                def _():
                    iwait()

                @pl.when(i >= NB)
                def _():
                    swait(b)

                ids = idx_v.at[pl.ds(sp * SUPI + o * C, C)]
                pltpu.async_copy(table_hbm.at[ids], rows[b], gsem[b])

                @pl.when(i >= L)
                def _():
                    q = (b - L) % NB
                    gwait(q)
                    store(i - L, q)

                @pl.when(jnp.logical_and(at_sup, s + 1 < n_super))
                def _():
                    nsp = 1 - sp
                    pltpu.async_copy(
                        x_hbm.at[pl.ds((base + (s + 1) * SUP * SC) * S, SUPI)],
                        idx_v.at[pl.ds(nsp * SUPI, SUPI)], isem)
            return carry

        lax.fori_loop(0, n_chunks // NB, body, 0)
        for t in range(L):
            j = n_chunks - L + t
            q = j % NB
            gwait(q)
            store(j, q)
        for b in range(NB):
            swait(b)

    return k


def kernel(x, table):
    R, S = x.shape
    V, D = table.shape
    P = 2
    Rp = R // P
    pieces = []
    for p in range(P):
        xf = x[p * Rp:(p + 1) * Rp].reshape(Rp * S).astype(jnp.int32)
        pieces.append(_make(Rp, S, V, D)(xf, table))
    return jnp.concatenate(pieces, axis=0)


# speedup vs baseline: 5.2553x; 1.0375x over previous
import functools

import jax
import jax.numpy as jnp
from jax import lax
from jax.experimental import pallas as pl
from jax.experimental.pallas import tpu as pltpu
from jax.experimental.pallas import tpu_sc as plsc


@functools.cache
def _make(R, S, V, D):
    info = plsc.get_sparse_core_info()
    NW = info.num_cores * info.num_subcores
    r_per_w = R // NW
    SC = 4
    C = SC * S
    n_chunks = r_per_w // SC
    SUP = 8
    SUPI = SUP * C
    n_super = n_chunks // SUP
    NB = 4
    L = NB - 1
    mesh = plsc.VectorSubcoreMesh(core_axis_name="c", subcore_axis_name="s")

    @functools.partial(
        pl.kernel,
        mesh=mesh,
        compiler_params=pltpu.CompilerParams(use_tc_tiling_on_sc=False),
        out_type=jax.ShapeDtypeStruct((R, S, D), jnp.float32),
        scratch_types=[
            pltpu.VMEM((2 * SUPI,), jnp.int32),
            *[pltpu.VMEM((C, D), jnp.float32) for _ in range(NB)],
            *[pltpu.SemaphoreType.DMA for _ in range(NB)],
            *[pltpu.SemaphoreType.DMA for _ in range(NB)],
            pltpu.SemaphoreType.DMA,
        ],
    )
    def k(x_hbm, table_hbm, out_hbm, idx_v, *refs):
        rows = refs[:NB]
        gsem = refs[NB:2 * NB]
        ssem = refs[2 * NB:3 * NB]
        isem = refs[3 * NB]
        wid = lax.axis_index("s") * info.num_cores + lax.axis_index("c")
        base = wid * r_per_w

        def store(i, p):
            for t in range(SC):
                pltpu.async_copy(
                    rows[p].at[pl.ds(t * S, S)],
                    out_hbm.at[base + i * SC + t], ssem[p])

        def gwait(p):
            pltpu.make_async_copy(table_hbm.at[pl.ds(0, C)], rows[p], gsem[p]).wait()

        def swait(p):
            for _ in range(SC):
                pltpu.make_async_copy(
                    rows[p].at[pl.ds(0, S)], out_hbm.at[0], ssem[p]).wait()

        def iwait():
            pltpu.make_async_copy(
                x_hbm.at[pl.ds(0, SUPI)], idx_v.at[pl.ds(0, SUPI)], isem).wait()

        pltpu.sync_copy(
            x_hbm.at[pl.ds(base * S, SUPI)], idx_v.at[pl.ds(0, SUPI)])

        def body(io, carry):
            for b in range(NB):
                i = io * NB + b
                s = i // SUP
                o = i - s * SUP
                sp = lax.rem(s, 2)
                at_sup = o == 0

                @pl.when(jnp.logical_and(at_sup, s >= 1))
                def _():
                    iwait()

                @pl.when(i >= NB)
                def _():
                    swait(b)

                ids = idx_v.at[pl.ds(sp * SUPI + o * C, C)]
                pltpu.async_copy(table_hbm.at[ids], rows[b], gsem[b])

                @pl.when(i >= L)
                def _():
                    q = (b - L) % NB
                    gwait(q)
                    store(i - L, q)

                @pl.when(jnp.logical_and(at_sup, s + 1 < n_super))
                def _():
                    nsp = 1 - sp
                    pltpu.async_copy(
                        x_hbm.at[pl.ds((base + (s + 1) * SUP * SC) * S, SUPI)],
                        idx_v.at[pl.ds(nsp * SUPI, SUPI)], isem)
            return carry

        lax.fori_loop(0, n_chunks // NB, body, 0)
        for t in range(L):
            j = n_chunks - L + t
            q = j % NB
            gwait(q)
            store(j, q)
        for b in range(NB):
            swait(b)

    return k


def kernel(x, table):
    R, S = x.shape
    V, D = table.shape
    P = 4
    Rp = R // P
    pieces = []
    for p in range(P):
        xf = x[p * Rp:(p + 1) * Rp].reshape(Rp * S).astype(jnp.int32)
        pieces.append(_make(Rp, S, V, D)(xf, table))
    return jnp.concatenate(pieces, axis=0)


# speedup vs baseline: 5.5970x; 1.0650x over previous
import functools

import jax
import jax.numpy as jnp
from jax import lax
from jax.experimental import pallas as pl
from jax.experimental.pallas import tpu as pltpu
from jax.experimental.pallas import tpu_sc as plsc


@functools.cache
def _make(R, S, V, D):
    info = plsc.get_sparse_core_info()
    NW = info.num_cores * info.num_subcores
    r_per_w = R // NW
    SC = 4
    C = SC * S
    n_chunks = r_per_w // SC
    SUP = 8
    SUPI = SUP * C
    n_super = n_chunks // SUP
    NB = 4
    L = NB - 1
    mesh = plsc.VectorSubcoreMesh(core_axis_name="c", subcore_axis_name="s")

    @functools.partial(
        pl.kernel,
        mesh=mesh,
        compiler_params=pltpu.CompilerParams(use_tc_tiling_on_sc=False),
        out_type=jax.ShapeDtypeStruct((R, S, D), jnp.float32),
        scratch_types=[
            pltpu.VMEM((2 * SUPI,), jnp.int32),
            *[pltpu.VMEM((C, D), jnp.float32) for _ in range(NB)],
            *[pltpu.SemaphoreType.DMA for _ in range(NB)],
            *[pltpu.SemaphoreType.DMA for _ in range(NB)],
            pltpu.SemaphoreType.DMA,
        ],
    )
    def k(x_hbm, table_hbm, out_hbm, idx_v, *refs):
        rows = refs[:NB]
        gsem = refs[NB:2 * NB]
        ssem = refs[2 * NB:3 * NB]
        isem = refs[3 * NB]
        wid = lax.axis_index("s") * info.num_cores + lax.axis_index("c")
        base = wid * r_per_w

        def store(i, p):
            for t in range(SC):
                pltpu.async_copy(
                    rows[p].at[pl.ds(t * S, S)],
                    out_hbm.at[base + i * SC + t], ssem[p])

        def gwait(p):
            pltpu.make_async_copy(table_hbm.at[pl.ds(0, C)], rows[p], gsem[p]).wait()

        def swait(p):
            for _ in range(SC):
                pltpu.make_async_copy(
                    rows[p].at[pl.ds(0, S)], out_hbm.at[0], ssem[p]).wait()

        def iwait():
            pltpu.make_async_copy(
                x_hbm.at[pl.ds(0, SUPI)], idx_v.at[pl.ds(0, SUPI)], isem).wait()

        pltpu.sync_copy(
            x_hbm.at[pl.ds(base * S, SUPI)], idx_v.at[pl.ds(0, SUPI)])

        def body(io, carry):
            for b in range(NB):
                i = io * NB + b
                s = i // SUP
                o = i - s * SUP
                sp = lax.rem(s, 2)
                at_sup = o == 0

                @pl.when(jnp.logical_and(at_sup, s >= 1))
                def _():
                    iwait()

                @pl.when(i >= NB)
                def _():
                    swait(b)

                ids = idx_v.at[pl.ds(sp * SUPI + o * C, C)]
                pltpu.async_copy(table_hbm.at[ids], rows[b], gsem[b])

                @pl.when(i >= L)
                def _():
                    q = (b - L) % NB
                    gwait(q)
                    store(i - L, q)

                @pl.when(jnp.logical_and(at_sup, s + 1 < n_super))
                def _():
                    nsp = 1 - sp
                    pltpu.async_copy(
                        x_hbm.at[pl.ds((base + (s + 1) * SUP * SC) * S, SUPI)],
                        idx_v.at[pl.ds(nsp * SUPI, SUPI)], isem)
            return carry

        lax.fori_loop(0, n_chunks // NB, body, 0)
        for t in range(L):
            j = n_chunks - L + t
            q = j % NB
            gwait(q)
            store(j, q)
        for b in range(NB):
            swait(b)

    return k


def kernel(x, table):
    R, S = x.shape
    V, D = table.shape
    P = 8
    Rp = R // P
    pieces = []
    for p in range(P):
        xf = x[p * Rp:(p + 1) * Rp].reshape(Rp * S).astype(jnp.int32)
        pieces.append(_make(Rp, S, V, D)(xf, table))
    return jnp.concatenate(pieces, axis=0)
